# baseline (speedup 1.0000x reference)
import functools

import jax
import jax.numpy as jnp
from jax import lax
from jax.experimental import pallas as pl
from jax.experimental.pallas import tpu as pltpu
from jax.experimental.pallas import tpu_sc as plsc

NC = 2
NS = 16
NW = NC * NS
LANES = 16

B_, S_, D_, V_ = 4, 8192, 1024, 64
R_ = B_ * S_
RPW = R_ // NW
C_ = 8
NCHUNK = RPW // C_
KV = D_ // LANES


def _sc_body(x_hbm, idx_hbm, table_hbm, out_hbm,
             table_v, idx_v, ib0, ib1, ob0, ob1,
             in_s0, in_s1, out_s0, out_s1):
    wid = lax.axis_index("s") * NC + lax.axis_index("c")
    base = wid * RPW

    pltpu.sync_copy(table_hbm, table_v)
    pltpu.sync_copy(idx_hbm.at[pl.ds(base, RPW)], idx_v.at[pl.ds(0, RPW)])

    ibufs = (ib0, ib1)
    obufs = (ob0, ob1)
    in_sems = (in_s0, in_s1)
    out_sems = (out_s0, out_s1)

    pltpu.async_copy(x_hbm.at[pl.ds(base, C_)], ib0, in_s0)
    pltpu.async_copy(x_hbm.at[pl.ds(base + C_, C_)], ib1, in_s1)

    def outer(g0, carry):
        for b in range(2):
            g = g0 * 2 + b
            row0 = base + g * C_
            ib, ob, isem, osem = ibufs[b], obufs[b], in_sems[b], out_sems[b]

            pltpu.make_async_copy(x_hbm.at[pl.ds(row0, C_)], ib, isem).wait()

            @pl.when(g >= 2)
            def _():
                pltpu.make_async_copy(
                    ob, out_hbm.at[pl.ds(row0 - 2 * C_, C_)], osem).wait()

            def row_body(r, c):
                t = idx_v[pl.ds(g * C_ + r, LANES)][0]
                for k in range(KV):
                    sl = pl.ds(k * LANES, LANES)
                    ob[r, sl] = ib[r, sl] + table_v[t, sl]
                return c
            lax.fori_loop(0, C_, row_body, 0)

            pltpu.async_copy(ob, out_hbm.at[pl.ds(row0, C_)], osem)

            @pl.when(g + 2 < NCHUNK)
            def _():
                pltpu.async_copy(x_hbm.at[pl.ds(row0 + 2 * C_, C_)], ib, isem)
        return carry

    lax.fori_loop(0, NCHUNK // 2, outer, 0)

    for b in range(2):
        g = NCHUNK - 2 + b
        row0 = base + g * C_
        pltpu.make_async_copy(
            obufs[b], out_hbm.at[pl.ds(row0, C_)], out_sems[b]).wait()


@functools.partial(jax.jit, static_argnames=())
def _sc_call(x2, idx, table):
    mesh = plsc.VectorSubcoreMesh(
        core_axis_name="c", subcore_axis_name="s",
        num_cores=NC, num_subcores=NS)
    return pl.kernel(
        _sc_body,
        out_type=jax.ShapeDtypeStruct((R_, D_), jnp.float32),
        mesh=mesh,
        scratch_types=[
            pltpu.VMEM((V_, D_), jnp.float32),
            pltpu.VMEM((RPW + LANES,), jnp.int32),
            pltpu.VMEM((C_, D_), jnp.float32),
            pltpu.VMEM((C_, D_), jnp.float32),
            pltpu.VMEM((C_, D_), jnp.float32),
            pltpu.VMEM((C_, D_), jnp.float32),
            pltpu.SemaphoreType.DMA,
            pltpu.SemaphoreType.DMA,
            pltpu.SemaphoreType.DMA,
            pltpu.SemaphoreType.DMA,
        ],
    )(x2, idx, table)


def kernel(x, indices, table):
    x2 = x.reshape(R_, D_)
    idx = indices.reshape(R_).astype(jnp.int32)
    out = _sc_call(x2, idx, table)
    return out.reshape(B_, S_, D_)


# speedup vs baseline: 2.1864x; 2.1864x over previous
import functools

import jax
import jax.numpy as jnp
from jax import lax
from jax.experimental import pallas as pl
from jax.experimental.pallas import tpu as pltpu
from jax.experimental.pallas import tpu_sc as plsc

NC = 2
NS = 16
NW = NC * NS
LANES = 16

B_, S_, D_, V_ = 4, 8192, 1024, 64
R_ = B_ * S_
RPW = R_ // NW
C_ = 8
NCHUNK = RPW // C_
KV = D_ // LANES


def _sc_body(x_hbm, idx_hbm, table_hbm, out_hbm,
             table_v, idx_v, ib0, ib1, ob0, ob1,
             in_s0, in_s1, out_s0, out_s1):
    wid = lax.axis_index("s") * NC + lax.axis_index("c")
    base = wid * RPW

    pltpu.sync_copy(table_hbm, table_v)
    pltpu.sync_copy(idx_hbm.at[pl.ds(base, RPW)], idx_v.at[pl.ds(0, RPW)])

    ibufs = (ib0, ib1)
    obufs = (ob0, ob1)
    in_sems = (in_s0, in_s1)
    out_sems = (out_s0, out_s1)

    pltpu.async_copy(x_hbm.at[pl.ds(base, C_)], ib0, in_s0)
    pltpu.async_copy(x_hbm.at[pl.ds(base + C_, C_)], ib1, in_s1)

    def outer(g0, carry):
        for b in range(2):
            g = g0 * 2 + b
            row0 = base + g * C_
            ib, ob, isem, osem = ibufs[b], obufs[b], in_sems[b], out_sems[b]

            pltpu.make_async_copy(x_hbm.at[pl.ds(row0, C_)], ib, isem).wait()

            @pl.when(g >= 2)
            def _():
                pltpu.make_async_copy(
                    ob, out_hbm.at[pl.ds(row0 - 2 * C_, C_)], osem).wait()

            def row_body(r, c):
                t = idx_v[pl.ds(g * C_ + r, LANES)][0]
                G = 8
                for k0 in range(0, KV, G):
                    sls = [pl.ds((k0 + j) * LANES, LANES) for j in range(G)]
                    tvs = [table_v[t, sls[j]] for j in range(G)]
                    xs = [ib[r, sls[j]] for j in range(G)]
                    for j in range(G):
                        ob[r, sls[j]] = xs[j] + tvs[j]
                return c
            lax.fori_loop(0, C_, row_body, 0)

            pltpu.async_copy(ob, out_hbm.at[pl.ds(row0, C_)], osem)

            @pl.when(g + 2 < NCHUNK)
            def _():
                pltpu.async_copy(x_hbm.at[pl.ds(row0 + 2 * C_, C_)], ib, isem)
        return carry

    lax.fori_loop(0, NCHUNK // 2, outer, 0)

    for b in range(2):
        g = NCHUNK - 2 + b
        row0 = base + g * C_
        pltpu.make_async_copy(
            obufs[b], out_hbm.at[pl.ds(row0, C_)], out_sems[b]).wait()


@functools.partial(jax.jit, static_argnames=())
def _sc_call(x2, idx, table):
    mesh = plsc.VectorSubcoreMesh(
        core_axis_name="c", subcore_axis_name="s",
        num_cores=NC, num_subcores=NS)
    return pl.kernel(
        _sc_body,
        out_type=jax.ShapeDtypeStruct((R_, D_), jnp.float32),
        mesh=mesh,
        scratch_types=[
            pltpu.VMEM((V_, D_), jnp.float32),
            pltpu.VMEM((RPW + LANES,), jnp.int32),
            pltpu.VMEM((C_, D_), jnp.float32),
            pltpu.VMEM((C_, D_), jnp.float32),
            pltpu.VMEM((C_, D_), jnp.float32),
            pltpu.VMEM((C_, D_), jnp.float32),
            pltpu.SemaphoreType.DMA,
            pltpu.SemaphoreType.DMA,
            pltpu.SemaphoreType.DMA,
            pltpu.SemaphoreType.DMA,
        ],
    )(x2, idx, table)


def kernel(x, indices, table):
    x2 = x.reshape(R_, D_)
    idx = indices.reshape(R_).astype(jnp.int32)
    out = _sc_call(x2, idx, table)
    return out.reshape(B_, S_, D_)


# speedup vs baseline: 2.1969x; 1.0048x over previous
import functools

import jax
import jax.numpy as jnp
from jax import lax
from jax.experimental import pallas as pl
from jax.experimental.pallas import tpu as pltpu
from jax.experimental.pallas import tpu_sc as plsc

NC = 2
NS = 16
NW = NC * NS
LANES = 16

B_, S_, D_, V_ = 4, 8192, 1024, 64
R_ = B_ * S_
RPW = R_ // NW
C_ = 8
NCHUNK = RPW // C_
KV = D_ // LANES
NBUF = 4


def _sc_body(x_hbm, idx_hbm, table_hbm, out_hbm,
             table_v, idx_v, ib0, ib1, ib2, ib3,
             is0, is1, is2, is3, os0, os1, os2, os3):
    wid = lax.axis_index("s") * NC + lax.axis_index("c")
    base = wid * RPW

    pltpu.sync_copy(table_hbm, table_v)
    pltpu.sync_copy(idx_hbm.at[pl.ds(base, RPW)], idx_v.at[pl.ds(0, RPW)])

    ibufs = (ib0, ib1, ib2, ib3)
    in_sems = (is0, is1, is2, is3)
    out_sems = (os0, os1, os2, os3)

    pltpu.async_copy(x_hbm.at[pl.ds(base, C_)], ib0, is0)
    pltpu.async_copy(x_hbm.at[pl.ds(base + C_, C_)], ib1, is1)

    def outer(g0, carry):
        for b in range(NBUF):
            g = g0 * NBUF + b
            row0 = base + g * C_
            ib = ibufs[b]

            pltpu.make_async_copy(x_hbm.at[pl.ds(row0, C_)], ib, in_sems[b]).wait()

            def row_body(r, c):
                t = idx_v[pl.ds(g * C_ + r, LANES)][0]
                G = 8
                for k0 in range(0, KV, G):
                    sls = [pl.ds((k0 + j) * LANES, LANES) for j in range(G)]
                    tvs = [table_v[t, sls[j]] for j in range(G)]
                    for j in range(G):
                        plsc.addupdate(ib.at[r, sls[j]], tvs[j])
                return c
            lax.fori_loop(0, C_, row_body, 0)

            pltpu.async_copy(ib, out_hbm.at[pl.ds(row0, C_)], out_sems[b])

            b2 = (b + 2) % NBUF
            ib2_ = ibufs[b2]

            @pl.when(g >= 2)
            def _():
                pltpu.make_async_copy(
                    ib2_, out_hbm.at[pl.ds(row0 - 2 * C_, C_)], out_sems[b2]).wait()

            @pl.when(g + 2 < NCHUNK)
            def _():
                pltpu.async_copy(
                    x_hbm.at[pl.ds(row0 + 2 * C_, C_)], ib2_, in_sems[b2])
        return carry

    lax.fori_loop(0, NCHUNK // NBUF, outer, 0)

    for g in (NCHUNK - 2, NCHUNK - 1):
        b = g % NBUF
        row0 = base + g * C_
        pltpu.make_async_copy(
            ibufs[b], out_hbm.at[pl.ds(row0, C_)], out_sems[b]).wait()


@functools.partial(jax.jit, static_argnames=())
def _sc_call(x2, idx, table):
    mesh = plsc.VectorSubcoreMesh(
        core_axis_name="c", subcore_axis_name="s",
        num_cores=NC, num_subcores=NS)
    return pl.kernel(
        _sc_body,
        out_type=jax.ShapeDtypeStruct((R_, D_), jnp.float32),
        mesh=mesh,
        scratch_types=[
            pltpu.VMEM((V_, D_), jnp.float32),
            pltpu.VMEM((RPW + LANES,), jnp.int32),
            pltpu.VMEM((C_, D_), jnp.float32),
            pltpu.VMEM((C_, D_), jnp.float32),
            pltpu.VMEM((C_, D_), jnp.float32),
            pltpu.VMEM((C_, D_), jnp.float32),
            pltpu.SemaphoreType.DMA,
            pltpu.SemaphoreType.DMA,
            pltpu.SemaphoreType.DMA,
            pltpu.SemaphoreType.DMA,
            pltpu.SemaphoreType.DMA,
            pltpu.SemaphoreType.DMA,
            pltpu.SemaphoreType.DMA,
            pltpu.SemaphoreType.DMA,
        ],
    )(x2, idx, table)


def kernel(x, indices, table):
    x2 = x.reshape(R_, D_)
    idx = indices.reshape(R_).astype(jnp.int32)
    out = _sc_call(x2, idx, table)
    return out.reshape(B_, S_, D_)


# speedup vs baseline: 2.7950x; 1.2722x over previous
import functools

import jax
import jax.numpy as jnp
from jax import lax
from jax.experimental import pallas as pl
from jax.experimental.pallas import tpu as pltpu
from jax.experimental.pallas import tpu_sc as plsc

NC = 2
NS = 16
NW = NC * NS
LANES = 16

B_, S_, D_, V_ = 4, 8192, 1024, 64
R_ = B_ * S_
RPW = R_ // NW
C_ = 8
NCHUNK = RPW // C_
KV = D_ // LANES
NBUF = 6
LOOK = 4


def _sc_body(x_hbm, idx_hbm, table_hbm, out_hbm,
             table_v, idx_v, ib0, ib1, ib2, ib3, ib4, ib5,
             is0, is1, is2, is3, is4, is5,
             os0, os1, os2, os3, os4, os5):
    wid = lax.axis_index("s") * NC + lax.axis_index("c")
    base = wid * RPW

    ibufs = (ib0, ib1, ib2, ib3, ib4, ib5)
    in_sems = (is0, is1, is2, is3, is4, is5)
    out_sems = (os0, os1, os2, os3, os4, os5)

    for j in range(LOOK):
        pltpu.async_copy(x_hbm.at[pl.ds(base + j * C_, C_)], ibufs[j], in_sems[j])
    pltpu.sync_copy(table_hbm, table_v)
    pltpu.sync_copy(idx_hbm.at[pl.ds(base, RPW)], idx_v.at[pl.ds(0, RPW)])

    def chunk_body(g, b, tail):
        row0 = base + g * C_
        ib = ibufs[b]

        pltpu.make_async_copy(x_hbm.at[pl.ds(row0, C_)], ib, in_sems[b]).wait()

        br = (b + LOOK) % NBUF
        ibr = ibufs[br]

        @pl.when(g >= NBUF - LOOK)
        def _():
            pltpu.make_async_copy(
                ibr, out_hbm.at[pl.ds(row0 - (NBUF - LOOK) * C_, C_)],
                out_sems[br]).wait()

        if not tail:
            @pl.when(g + LOOK < NCHUNK)
            def _():
                pltpu.async_copy(
                    x_hbm.at[pl.ds(row0 + LOOK * C_, C_)], ibr, in_sems[br])

        def row_body(r, c):
            t = idx_v[pl.ds(g * C_ + r, LANES)][0]
            G = 8
            for k0 in range(0, KV, G):
                sls = [pl.ds((k0 + j) * LANES, LANES) for j in range(G)]
                tvs = [table_v[t, sls[j]] for j in range(G)]
                for j in range(G):
                    plsc.addupdate(ib.at[r, sls[j]], tvs[j])
            return c
        lax.fori_loop(0, C_, row_body, 0)

        pltpu.async_copy(ib, out_hbm.at[pl.ds(row0, C_)], out_sems[b])

    NFULL = (NCHUNK // NBUF) * NBUF

    def outer(g0, carry):
        for b in range(NBUF):
            chunk_body(g0 * NBUF + b, b, tail=False)
        return carry

    lax.fori_loop(0, NFULL // NBUF, outer, 0)

    for g in range(NFULL, NCHUNK):
        chunk_body(g, g % NBUF, tail=True)

    for g in range(NCHUNK - (NBUF - LOOK), NCHUNK):
        b = g % NBUF
        row0 = base + g * C_
        pltpu.make_async_copy(
            ibufs[b], out_hbm.at[pl.ds(row0, C_)], out_sems[b]).wait()


@functools.partial(jax.jit, static_argnames=())
def _sc_call(x2, idx, table):
    mesh = plsc.VectorSubcoreMesh(
        core_axis_name="c", subcore_axis_name="s",
        num_cores=NC, num_subcores=NS)
    return pl.kernel(
        _sc_body,
        out_type=jax.ShapeDtypeStruct((R_, D_), jnp.float32),
        mesh=mesh,
        scratch_types=(
            [pltpu.VMEM((V_, D_), jnp.float32),
             pltpu.VMEM((RPW + LANES,), jnp.int32)]
            + [pltpu.VMEM((C_, D_), jnp.float32) for _ in range(NBUF)]
            + [pltpu.SemaphoreType.DMA for _ in range(2 * NBUF)]
        ),
    )(x2, idx, table)


def kernel(x, indices, table):
    x2 = x.reshape(R_, D_)
    idx = indices.reshape(R_).astype(jnp.int32)
    out = _sc_call(x2, idx, table)
    return out.reshape(B_, S_, D_)
